# baseline (speedup 1.0000x reference)
import functools

import jax
import jax.numpy as jnp
from jax import lax
from jax.experimental import pallas as pl
from jax.experimental.pallas import tpu as pltpu
from jax.experimental.pallas import tpu_sc as plsc

H = 512
W = 512
B = 4
P = 20000
C = 64
L = 16

_info = plsc.get_sparse_core_info()
NC = _info.num_cores
NS = _info.num_subcores
NT = NC * NS
RPT = H // NT
CELLS = RPT * W
BAND = 8
BCELLS = BAND * W
CAP = 2048
BCAP = 512
GCH = 64
GSH = 6
SRNG = 1248
SCH = 80

_mesh = plsc.VectorSubcoreMesh(core_axis_name="c", subcore_axis_name="s")


@functools.partial(
    pl.kernel,
    mesh=_mesh,
    compiler_params=pltpu.CompilerParams(needs_layout_passes=False,
                                        use_tc_tiling_on_sc=False),
    out_type=jax.ShapeDtypeStruct((B, C, H, W), jnp.float32),
    scratch_types=[
        pltpu.VMEM((SRNG + 32,), jnp.int32),
        pltpu.VMEM((SRNG + 32,), jnp.int32),
        pltpu.VMEM((NS * 128,), jnp.int32),
        pltpu.VMEM((NS * 128,), jnp.int32),
        pltpu.VMEM_SHARED((NS * NS * 128,), jnp.int32),
        pltpu.VMEM((CAP + L,), jnp.int32),
        pltpu.VMEM((BCAP + L,), jnp.int32),
        pltpu.VMEM((BCAP + L,), jnp.int32),
        pltpu.VMEM((CELLS,), jnp.int32),
        pltpu.VMEM((L,), jnp.int32),
        pltpu.VMEM((L,), jnp.int32),
        pltpu.VMEM((BCAP + L,), jnp.int32),
        pltpu.VMEM((BCAP + L,), jnp.int32),
        pltpu.VMEM((BCAP + L,), jnp.int32),
        pltpu.VMEM((BCAP + L,), jnp.int32),
        pltpu.VMEM((BCAP + L,), jnp.int32),
        pltpu.VMEM((BCAP + L,), jnp.int32),
        pltpu.VMEM((BCAP,), jnp.int32),
        pltpu.VMEM((BCAP,), jnp.int32),
        pltpu.VMEM((BCAP, C), jnp.float32),
        pltpu.VMEM((4, BAND, W), jnp.float32),
        pltpu.SemaphoreType.DMA,
        pltpu.SemaphoreType.DMA,
        pltpu.SemaphoreType.DMA,
        pltpu.SemaphoreType.DMA,
        pltpu.SemaphoreType.DMA,
        pltpu.SemaphoreType.DMA,
    ],
)
def _bev_sc(feats, gxh, gyh, zb, out, gxv, gyv, bktf, tmpb, shm, cpk,
            bpk0, bpk1, wm, spos, pkb, rowl0, rowl1, coll0, coll1, hafl0,
            hafl1, grow0, grow1, winsrc, bands,
            semc, semg, semb0, semb1, semb2, semb3):
    cc = lax.axis_index("c")
    ss = lax.axis_index("s")
    wid = cc * NS + ss
    r0 = (wid * RPT).astype(jnp.int32)
    ids = lax.iota(jnp.int32, L)
    zi = jnp.zeros((L,), jnp.int32)
    zf = jnp.zeros((L,), jnp.float32)
    sems = (semb0, semb1, semb2, semb3)
    bpk = (bpk0, bpk1)
    rowl = (rowl0, rowl1)
    coll = (coll0, coll1)
    hafl = (hafl0, hafl1)
    grow = (grow0, grow1)

    def _wm_body(j, carry):
        wm[pl.ds(j * L, L)] = zi
        return carry

    lax.fori_loop(0, CELLS // L, _wm_body, 0)
    for u in range(4):
        pltpu.sync_copy(zb, bands.at[u])

    def do_scan(b):
        stamp0 = jnp.int32(b * P + 1)
        base_pid = ss * SRNG
        lim = jnp.where(ss == NS - 1, jnp.int32(P - (NS - 1) * SRNG),
                        jnp.int32(SRNG))
        cbase = (cc * (NS * RPT)).astype(jnp.int32)

        pltpu.sync_copy(gxh.at[pl.ds(b * P + base_pid, SRNG + 32)], gxv)
        pltpu.sync_copy(gyh.at[pl.ds(b * P + base_pid, SRNG + 32)], gyv)

        plsc.store_scatter(bktf, [ids * 128], zi)

        def _route(ci, cnts):
            gyc = gyv[pl.ds(ci * L, L)]
            gxc = gxv[pl.ds(ci * L, L)]
            pidl = ci * L + ids
            m = (gyc >= cbase) & (gyc < cbase + NS * RPT) & (pidl < lim)
            posl = (gyc & (RPT - 1)) * W + gxc
            pk = posl * 32768 + (base_pid + pidl)
            ov = lax.shift_right_arithmetic(gyc, 4) - cc * NS
            new = []
            for o in range(NS):
                mo = m & (ov == o)
                plsc.store_compressed(
                    bktf.at[pl.ds(o * 128 + 1 + cnts[o], L)], pk, mask=mo)
                po = plsc.all_reduce_population_count(mo)
                new.append(jnp.minimum(cnts[o] + po[0], jnp.int32(111)))
            return tuple(new)

        with jax.named_scope("scan"):
            cnts = lax.fori_loop(0, SCH, _route,
                                 tuple(jnp.int32(0) for _ in range(NS)))

        cvec = zi
        for o in range(NS):
            cvec = jnp.where(ids == o, cnts[o], cvec)
        plsc.store_scatter(bktf, [ids * 128], cvec)
        plsc.subcore_barrier()
        pltpu.sync_copy(bktf, shm.at[pl.ds(ss * (NS * 128), NS * 128)])
        plsc.subcore_barrier()

        def _coll(t):
            return pltpu.make_async_copy(
                shm.at[pl.ds((t * NS + ss) * 128, 128)],
                tmpb.at[pl.ds(t * 128, 128)], semc)

        for t in range(NS):
            _coll(t).start()
        for t in range(NS):
            _coll(t).wait()

        counts = plsc.load_gather(tmpb, [ids * 128])
        incl = plsc.cumsum(counts)
        total = incl[NS - 1]

        with jax.named_scope("concat"):
            off = jnp.int32(0)
            for t in range(NS):
                for j in range(7):
                    cpk[pl.ds(off + j * L, L)] = tmpb[
                        pl.ds(t * 128 + 1 + j * L, L)]
                off = incl[t]

        def _dedup(k, carry):
            base = k * L
            ok = (base + ids) < total
            pk = cpk[pl.ds(base, L)]
            posl = lax.shift_right_arithmetic(pk, 15)
            key = posl * L + ids
            sk, _, sm = plsc.sort_key_val(key, key, mask=ok)
            pos_s = lax.shift_right_arithmetic(sk, 4)
            lane_s = sk & (L - 1)
            spos[:] = pos_s
            pkb[:] = pk
            nxt = plsc.load_gather(spos, [jnp.minimum(ids + 1, L - 1)])
            pk_s = plsc.load_gather(pkb, [lane_s])
            m_store = ((pos_s != nxt) | (ids == L - 1)) & sm
            val = stamp0 + (pk_s & 32767)
            plsc.store_scatter(wm, [pos_s], val, mask=m_store)
            return carry

        with jax.named_scope("dedup"):
            lax.fori_loop(0, lax.shift_right_arithmetic(total + L - 1, 4),
                          _dedup, 0)
        return total

    def do_ext(b, n):
        stamp0 = jnp.int32(b * P + 1)

        def _ext(j, nbs):
            nb0, nb1 = nbs
            base = j * L
            pk = cpk[pl.ds(base, L)]
            posc = lax.shift_right_arithmetic(pk, 15)
            idxc = pk & 32767
            ok = (base + ids) < n
            wv = plsc.load_gather(wm, [posc], mask=ok)
            win = ok & (wv == stamp0 + idxc)
            w0 = win & (posc < BCELLS)
            w1 = win & (posc >= BCELLS)
            plsc.store_compressed(bpk[0].at[pl.ds(nb0, L)], pk, mask=w0)
            plsc.store_compressed(bpk[1].at[pl.ds(nb1, L)],
                                  pk - BCELLS * 32768, mask=w1)
            c0 = plsc.all_reduce_population_count(w0)
            c1 = plsc.all_reduce_population_count(w1)
            return (jnp.minimum(nb0 + c0[0], jnp.int32(BCAP)),
                    jnp.minimum(nb1 + c1[0], jnp.int32(BCAP)))

        with jax.named_scope("ext"):
            return lax.fori_loop(
                0, lax.shift_right_arithmetic(n + L - 1, 4), _ext,
                (jnp.int32(0), jnp.int32(0)))

    def do_prep(b, band, nb):
        def _rc(k, carry):
            base = k * L
            pk = bpk[band][pl.ds(base, L)]
            posc = lax.shift_right_arithmetic(pk, 15)
            pid = pk & 32767
            ok = (base + ids) < nb
            rowl[band][pl.ds(base, L)] = lax.shift_right_arithmetic(
                posc, 9)
            coll[band][pl.ds(base, L)] = posc & (W - 1)
            grow[band][pl.ds(base, L)] = jnp.where(
                ok, jnp.int32(b * P) + pid, 0)
            return carry

        with jax.named_scope("prep"):
            nch = lax.shift_left(
                lax.shift_right_arithmetic(nb + GCH - 1, GSH), GSH - 4)
            lax.fori_loop(0, nch, _rc, 0)

    def _g_dma(band, g):
        return pltpu.make_async_copy(
            feats.at[grow[band].at[pl.ds(g * GCH, GCH)]],
            winsrc.at[pl.ds(g * GCH, GCH), :], semg)

    def fire_g(band, nb):
        def _gf(g, carry):
            _g_dma(band, g).start()
            return carry

        lax.fori_loop(0, lax.shift_right_arithmetic(nb + GCH - 1, GSH),
                      _gf, 0)

    def wait_g(band, nb):
        def _gw(g, carry):
            _g_dma(band, g).wait()
            return carry

        with jax.named_scope("rowwait"):
            lax.fori_loop(0, lax.shift_right_arithmetic(nb + GCH - 1, GSH),
                          _gw, 0)

    def do_bands(b, band, nb):
        row0 = r0 + band * BAND
        nk = lax.shift_right_arithmetic(nb + L - 1, 4)

        def _insert2(u0, c0):
            def _ins(k, carry):
                base = k * L
                kv = base + ids
                ok = kv < nb
                rw = rowl[band][pl.ds(base, L)]
                cl = coll[band][pl.ds(base, L)]
                v0 = plsc.load_gather(winsrc, [kv, c0 + zi], mask=ok)
                plsc.store_scatter(bands.at[u0], [rw, cl], v0, mask=ok)
                v1 = plsc.load_gather(winsrc, [kv, c0 + 1 + zi], mask=ok)
                plsc.store_scatter(bands.at[u0 + 1], [rw, cl], v1, mask=ok)
                return carry

            lax.fori_loop(0, nk, _ins, 0)

        def _band_dma(u, c_tr, sem):
            return pltpu.make_async_copy(
                bands.at[u], out.at[b, c_tr, pl.ds(row0, BAND), :], sem)

        for pp in range(2):
            c0 = jnp.int32(2 * pp)
            _insert2(2 * pp, c0)
            _band_dma(2 * pp, c0, sems[2 * pp]).start()
            _band_dma(2 * pp + 1, c0 + 1, sems[2 * pp + 1]).start()

        def _grp(g2, carry):
            for pp in range(2):
                c0 = (g2 * 2 + pp) * 2
                u0 = 2 * pp
                _band_dma(u0, c0 - 4, sems[u0]).wait()
                _band_dma(u0 + 1, c0 - 3, sems[u0 + 1]).wait()
                _insert2(u0, c0)
                _band_dma(u0, c0, sems[u0]).start()
                _band_dma(u0 + 1, c0 + 1, sems[u0 + 1]).start()
            return carry

        with jax.named_scope("bands2"):
            lax.fori_loop(1, C // 4, _grp, 0)

        for u in range(4):
            _band_dma(u, jnp.int32(C - 4 + u), sems[u]).wait()

            def _rst(k, carry):
                base = k * L
                ok = (base + ids) < nb
                rw = rowl[band][pl.ds(base, L)]
                cl = coll[band][pl.ds(base, L)]
                plsc.store_scatter(bands.at[u], [rw, cl], zf, mask=ok)
                return carry

            with jax.named_scope("rst"):
                lax.fori_loop(0, nk, _rst, 0)

    n = do_scan(0)
    nb0, nb1 = do_ext(0, n)
    do_prep(0, 0, nb0)
    do_prep(0, 1, nb1)
    fire_g(0, nb0)
    for b in range(B):
        if b + 1 < B:
            n = do_scan(b + 1)
        wait_g(0, nb0)
        do_bands(b, 0, nb0)
        fire_g(1, nb1)
        wait_g(1, nb1)
        do_bands(b, 1, nb1)
        if b + 1 < B:
            nb0n, nb1n = do_ext(b + 1, n)
            do_prep(b + 1, 0, nb0n)
            do_prep(b + 1, 1, nb1n)
            fire_g(0, nb0n)
            nb0, nb1 = nb0n, nb1n


def kernel(pillar_features, pillar_coords):
    gx = pillar_coords[..., 0].reshape(-1).astype(jnp.int32)
    gy = pillar_coords[..., 1].reshape(-1).astype(jnp.int32)
    feats = pillar_features.reshape(B * P, C)
    zb = jnp.zeros((BAND, W), jnp.float32)
    return _bev_sc(feats, gx, gy, zb)


# speedup vs baseline: 1.3691x; 1.3691x over previous
import functools

import jax
import jax.numpy as jnp
from jax import lax
from jax.experimental import pallas as pl
from jax.experimental.pallas import tpu as pltpu
from jax.experimental.pallas import tpu_sc as plsc

H = 512
W = 512
B = 4
P = 20000
C = 64
L = 16

_info = plsc.get_sparse_core_info()
NC = _info.num_cores
NS = _info.num_subcores
NT = NC * NS
RPT = H // NT
CELLS = RPT * W
BAND = 8
BCELLS = BAND * W
CAP = 2048
BCAP = 512
GCH = 32
GSH = 5
SRNG = 1248
SCH = 80

_mesh = plsc.VectorSubcoreMesh(core_axis_name="c", subcore_axis_name="s")


@functools.partial(
    pl.kernel,
    mesh=_mesh,
    compiler_params=pltpu.CompilerParams(needs_layout_passes=False),
    out_type=jax.ShapeDtypeStruct((B, C, H, W), jnp.float32),
    scratch_types=[
        pltpu.VMEM((SRNG + 32,), jnp.int32),
        pltpu.VMEM((SRNG + 32,), jnp.int32),
        pltpu.VMEM((NS * 128,), jnp.int32),
        pltpu.VMEM((NS * 128,), jnp.int32),
        pltpu.VMEM_SHARED((NS * NS * 128,), jnp.int32),
        pltpu.VMEM((CAP + L,), jnp.int32),
        pltpu.VMEM((BCAP + L,), jnp.int32),
        pltpu.VMEM((BCAP + L,), jnp.int32),
        pltpu.VMEM((CELLS,), jnp.int32),
        pltpu.VMEM((L,), jnp.int32),
        pltpu.VMEM((L,), jnp.int32),
        pltpu.VMEM((BCAP + L,), jnp.int32),
        pltpu.VMEM((BCAP + L,), jnp.int32),
        pltpu.VMEM((BCAP + L,), jnp.int32),
        pltpu.VMEM((BCAP + L,), jnp.int32),
        pltpu.VMEM((BCAP + L,), jnp.int32),
        pltpu.VMEM((BCAP + L,), jnp.int32),
        pltpu.VMEM((BCAP,), jnp.int32),
        pltpu.VMEM((BCAP,), jnp.int32),
        pltpu.VMEM((BCAP, 2 * C), jnp.float32),
        pltpu.VMEM((4, BAND, W), jnp.float32),
        pltpu.SemaphoreType.DMA,
        pltpu.SemaphoreType.DMA,
        pltpu.SemaphoreType.DMA,
        pltpu.SemaphoreType.DMA,
        pltpu.SemaphoreType.DMA,
        pltpu.SemaphoreType.DMA,
    ],
)
def _bev_sc(feats, gxh, gyh, zb, out, gxv, gyv, bktf, tmpb, shm, cpk,
            bpk0, bpk1, wm, spos, pkb, rowl0, rowl1, coll0, coll1, hafl0,
            hafl1, grow0, grow1, winsrc, bands,
            semc, semg, semb0, semb1, semb2, semb3):
    cc = lax.axis_index("c")
    ss = lax.axis_index("s")
    wid = cc * NS + ss
    r0 = (wid * RPT).astype(jnp.int32)
    ids = lax.iota(jnp.int32, L)
    zi = jnp.zeros((L,), jnp.int32)
    zf = jnp.zeros((L,), jnp.float32)
    sems = (semb0, semb1, semb2, semb3)
    bpk = (bpk0, bpk1)
    rowl = (rowl0, rowl1)
    coll = (coll0, coll1)
    hafl = (hafl0, hafl1)
    grow = (grow0, grow1)

    def _wm_body(j, carry):
        wm[pl.ds(j * L, L)] = zi
        return carry

    lax.fori_loop(0, CELLS // L, _wm_body, 0)
    for u in range(4):
        pltpu.sync_copy(zb, bands.at[u])

    def do_scan(b):
        stamp0 = jnp.int32(b * P + 1)
        base_pid = ss * SRNG
        lim = jnp.where(ss == NS - 1, jnp.int32(P - (NS - 1) * SRNG),
                        jnp.int32(SRNG))
        cbase = (cc * (NS * RPT)).astype(jnp.int32)

        pltpu.sync_copy(gxh.at[pl.ds(b * P + base_pid, SRNG + 32)], gxv)
        pltpu.sync_copy(gyh.at[pl.ds(b * P + base_pid, SRNG + 32)], gyv)

        plsc.store_scatter(bktf, [ids * 128], zi)

        def _route(ci, cnts):
            gyc = gyv[pl.ds(ci * L, L)]
            gxc = gxv[pl.ds(ci * L, L)]
            pidl = ci * L + ids
            m = (gyc >= cbase) & (gyc < cbase + NS * RPT) & (pidl < lim)
            posl = (gyc & (RPT - 1)) * W + gxc
            pk = posl * 32768 + (base_pid + pidl)
            ov = lax.shift_right_arithmetic(gyc, 4) - cc * NS
            new = []
            for o in range(NS):
                mo = m & (ov == o)
                plsc.store_compressed(
                    bktf.at[pl.ds(o * 128 + 1 + cnts[o], L)], pk, mask=mo)
                po = plsc.all_reduce_population_count(mo)
                new.append(jnp.minimum(cnts[o] + po[0], jnp.int32(111)))
            return tuple(new)

        with jax.named_scope("scan"):
            cnts = lax.fori_loop(0, SCH, _route,
                                 tuple(jnp.int32(0) for _ in range(NS)))

        cvec = zi
        for o in range(NS):
            cvec = jnp.where(ids == o, cnts[o], cvec)
        plsc.store_scatter(bktf, [ids * 128], cvec)
        plsc.subcore_barrier()
        pltpu.sync_copy(bktf, shm.at[pl.ds(ss * (NS * 128), NS * 128)])
        plsc.subcore_barrier()

        def _coll(t):
            return pltpu.make_async_copy(
                shm.at[pl.ds((t * NS + ss) * 128, 128)],
                tmpb.at[pl.ds(t * 128, 128)], semc)

        for t in range(NS):
            _coll(t).start()
        for t in range(NS):
            _coll(t).wait()

        counts = plsc.load_gather(tmpb, [ids * 128])
        incl = plsc.cumsum(counts)
        total = incl[NS - 1]

        with jax.named_scope("concat"):
            off = jnp.int32(0)
            for t in range(NS):
                for j in range(7):
                    cpk[pl.ds(off + j * L, L)] = tmpb[
                        pl.ds(t * 128 + 1 + j * L, L)]
                off = incl[t]

        def _dedup(k, carry):
            base = k * L
            ok = (base + ids) < total
            pk = cpk[pl.ds(base, L)]
            posl = lax.shift_right_arithmetic(pk, 15)
            key = posl * L + ids
            sk, _, sm = plsc.sort_key_val(key, key, mask=ok)
            pos_s = lax.shift_right_arithmetic(sk, 4)
            lane_s = sk & (L - 1)
            spos[:] = pos_s
            pkb[:] = pk
            nxt = plsc.load_gather(spos, [jnp.minimum(ids + 1, L - 1)])
            pk_s = plsc.load_gather(pkb, [lane_s])
            m_store = ((pos_s != nxt) | (ids == L - 1)) & sm
            val = stamp0 + (pk_s & 32767)
            plsc.store_scatter(wm, [pos_s], val, mask=m_store)
            return carry

        with jax.named_scope("dedup"):
            lax.fori_loop(0, lax.shift_right_arithmetic(total + L - 1, 4),
                          _dedup, 0)
        return total

    def do_ext(b, n):
        stamp0 = jnp.int32(b * P + 1)

        def _ext(j, nbs):
            nb0, nb1 = nbs
            base = j * L
            pk = cpk[pl.ds(base, L)]
            posc = lax.shift_right_arithmetic(pk, 15)
            idxc = pk & 32767
            ok = (base + ids) < n
            wv = plsc.load_gather(wm, [posc], mask=ok)
            win = ok & (wv == stamp0 + idxc)
            w0 = win & (posc < BCELLS)
            w1 = win & (posc >= BCELLS)
            plsc.store_compressed(bpk[0].at[pl.ds(nb0, L)], pk, mask=w0)
            plsc.store_compressed(bpk[1].at[pl.ds(nb1, L)],
                                  pk - BCELLS * 32768, mask=w1)
            c0 = plsc.all_reduce_population_count(w0)
            c1 = plsc.all_reduce_population_count(w1)
            return (jnp.minimum(nb0 + c0[0], jnp.int32(BCAP)),
                    jnp.minimum(nb1 + c1[0], jnp.int32(BCAP)))

        with jax.named_scope("ext"):
            return lax.fori_loop(
                0, lax.shift_right_arithmetic(n + L - 1, 4), _ext,
                (jnp.int32(0), jnp.int32(0)))

    def do_prep(b, band, nb):
        def _rc(k, carry):
            base = k * L
            pk = bpk[band][pl.ds(base, L)]
            posc = lax.shift_right_arithmetic(pk, 15)
            pid = pk & 32767
            ok = (base + ids) < nb
            rowl[band][pl.ds(base, L)] = lax.shift_right_arithmetic(
                posc, 9)
            coll[band][pl.ds(base, L)] = posc & (W - 1)
            hafl[band][pl.ds(base, L)] = (pid & 1) * C
            grow[band][pl.ds(base, L)] = jnp.where(
                ok, lax.shift_right_arithmetic(jnp.int32(b * P) + pid, 1),
                0)
            return carry

        with jax.named_scope("prep"):
            nch = lax.shift_left(
                lax.shift_right_arithmetic(nb + GCH - 1, GSH), GSH - 4)
            lax.fori_loop(0, nch, _rc, 0)

    def _g_dma(band, g):
        return pltpu.make_async_copy(
            feats.at[grow[band].at[pl.ds(g * GCH, GCH)]],
            winsrc.at[pl.ds(g * GCH, GCH), :], semg)

    def fire_g(band, nb):
        def _gf(g, carry):
            _g_dma(band, g).start()
            return carry

        lax.fori_loop(0, lax.shift_right_arithmetic(nb + GCH - 1, GSH),
                      _gf, 0)

    def wait_g(band, nb):
        def _gw(g, carry):
            _g_dma(band, g).wait()
            return carry

        with jax.named_scope("rowwait"):
            lax.fori_loop(0, lax.shift_right_arithmetic(nb + GCH - 1, GSH),
                          _gw, 0)

    def do_bands(b, band, nb):
        row0 = r0 + band * BAND
        nk = lax.shift_right_arithmetic(nb + L - 1, 4)

        def _insert2(u0, c0):
            def _ins(k, carry):
                base = k * L
                kv = base + ids
                ok = kv < nb
                rw = rowl[band][pl.ds(base, L)]
                cl = coll[band][pl.ds(base, L)]
                hf = hafl[band][pl.ds(base, L)]
                v0 = plsc.load_gather(winsrc, [kv, hf + c0], mask=ok)
                plsc.store_scatter(bands.at[u0], [rw, cl], v0, mask=ok)
                v1 = plsc.load_gather(winsrc, [kv, hf + (c0 + 1)], mask=ok)
                plsc.store_scatter(bands.at[u0 + 1], [rw, cl], v1, mask=ok)
                return carry

            lax.fori_loop(0, nk, _ins, 0)

        def _band_dma(u, c_tr, sem):
            return pltpu.make_async_copy(
                bands.at[u], out.at[b, c_tr, pl.ds(row0, BAND), :], sem)

        for pp in range(2):
            c0 = jnp.int32(2 * pp)
            _insert2(2 * pp, c0)
            _band_dma(2 * pp, c0, sems[2 * pp]).start()
            _band_dma(2 * pp + 1, c0 + 1, sems[2 * pp + 1]).start()

        def _grp(g2, carry):
            for pp in range(2):
                c0 = (g2 * 2 + pp) * 2
                u0 = 2 * pp
                _band_dma(u0, c0 - 4, sems[u0]).wait()
                _band_dma(u0 + 1, c0 - 3, sems[u0 + 1]).wait()
                _insert2(u0, c0)
                _band_dma(u0, c0, sems[u0]).start()
                _band_dma(u0 + 1, c0 + 1, sems[u0 + 1]).start()
            return carry

        with jax.named_scope("bands2"):
            lax.fori_loop(1, C // 4, _grp, 0)

        for u in range(4):
            _band_dma(u, jnp.int32(C - 4 + u), sems[u]).wait()

            def _rst(k, carry):
                base = k * L
                ok = (base + ids) < nb
                rw = rowl[band][pl.ds(base, L)]
                cl = coll[band][pl.ds(base, L)]
                plsc.store_scatter(bands.at[u], [rw, cl], zf, mask=ok)
                return carry

            with jax.named_scope("rst"):
                lax.fori_loop(0, nk, _rst, 0)

    n = do_scan(0)
    nb0, nb1 = do_ext(0, n)
    do_prep(0, 0, nb0)
    do_prep(0, 1, nb1)
    fire_g(0, nb0)
    for b in range(B):
        if b + 1 < B:
            n = do_scan(b + 1)
        wait_g(0, nb0)
        do_bands(b, 0, nb0)
        fire_g(1, nb1)
        wait_g(1, nb1)
        do_bands(b, 1, nb1)
        if b + 1 < B:
            nb0n, nb1n = do_ext(b + 1, n)
            do_prep(b + 1, 0, nb0n)
            do_prep(b + 1, 1, nb1n)
            fire_g(0, nb0n)
            nb0, nb1 = nb0n, nb1n


def kernel(pillar_features, pillar_coords):
    gx = pillar_coords[..., 0].reshape(-1).astype(jnp.int32)
    gy = pillar_coords[..., 1].reshape(-1).astype(jnp.int32)
    feats = pillar_features.reshape(B * P // 2, 2 * C)
    zb = jnp.zeros((BAND, W), jnp.float32)
    return _bev_sc(feats, gx, gy, zb)


# speedup vs baseline: 1.8127x; 1.3240x over previous
import functools

import jax
import jax.numpy as jnp
from jax import lax
from jax.experimental import pallas as pl
from jax.experimental.pallas import tpu as pltpu
from jax.experimental.pallas import tpu_sc as plsc

H = 512
W = 512
B = 4
P = 20000
C = 64
L = 16

_info = plsc.get_sparse_core_info()
NC = _info.num_cores
NS = _info.num_subcores
NT = NC * NS
RPT = H // NT
CELLS = RPT * W
BAND = 8
BCELLS = BAND * W
CAP = 2048
BCAP = 512
GCH = 16
GSH = 4
SRNG = 1248
SCH = 80

_mesh = plsc.VectorSubcoreMesh(core_axis_name="c", subcore_axis_name="s")


@functools.partial(
    pl.kernel,
    mesh=_mesh,
    compiler_params=pltpu.CompilerParams(needs_layout_passes=False),
    out_type=jax.ShapeDtypeStruct((B, C, H, W), jnp.float32),
    scratch_types=[
        pltpu.VMEM((SRNG + 32,), jnp.int32),
        pltpu.VMEM((SRNG + 32,), jnp.int32),
        pltpu.VMEM((NS * 128,), jnp.int32),
        pltpu.VMEM((NS * 128,), jnp.int32),
        pltpu.VMEM_SHARED((NS * NS * 128,), jnp.int32),
        pltpu.VMEM((CAP + L,), jnp.int32),
        pltpu.VMEM((BCAP + L,), jnp.int32),
        pltpu.VMEM((BCAP + L,), jnp.int32),
        pltpu.VMEM((CELLS,), jnp.int32),
        pltpu.VMEM((L,), jnp.int32),
        pltpu.VMEM((L,), jnp.int32),
        pltpu.VMEM((BCAP + L,), jnp.int32),
        pltpu.VMEM((BCAP + L,), jnp.int32),
        pltpu.VMEM((BCAP + L,), jnp.int32),
        pltpu.VMEM((BCAP + L,), jnp.int32),
        pltpu.VMEM((BCAP + L,), jnp.int32),
        pltpu.VMEM((BCAP + L,), jnp.int32),
        pltpu.VMEM((BCAP,), jnp.int32),
        pltpu.VMEM((BCAP,), jnp.int32),
        pltpu.VMEM((BCAP, 2 * C), jnp.float32),
        pltpu.VMEM((4, BAND, W), jnp.float32),
        pltpu.SemaphoreType.DMA,
        pltpu.SemaphoreType.DMA,
        pltpu.SemaphoreType.DMA,
        pltpu.SemaphoreType.DMA,
        pltpu.SemaphoreType.DMA,
        pltpu.SemaphoreType.DMA,
    ],
)
def _bev_sc(feats, gxh, gyh, zb, out, gxv, gyv, bktf, tmpb, shm, cpk,
            bpk0, bpk1, wm, spos, pkb, rowl0, rowl1, coll0, coll1, hafl0,
            hafl1, grow0, grow1, winsrc, bands,
            semc, semg, semb0, semb1, semb2, semb3):
    cc = lax.axis_index("c")
    ss = lax.axis_index("s")
    wid = cc * NS + ss
    r0 = (wid * RPT).astype(jnp.int32)
    ids = lax.iota(jnp.int32, L)
    zi = jnp.zeros((L,), jnp.int32)
    zf = jnp.zeros((L,), jnp.float32)
    sems = (semb0, semb1, semb2, semb3)
    bpk = (bpk0, bpk1)
    rowl = (rowl0, rowl1)
    coll = (coll0, coll1)
    hafl = (hafl0, hafl1)
    grow = (grow0, grow1)

    def _wm_body(j, carry):
        wm[pl.ds(j * L, L)] = zi
        return carry

    lax.fori_loop(0, CELLS // L, _wm_body, 0)
    for u in range(4):
        pltpu.sync_copy(zb, bands.at[u])

    def do_scan(b):
        stamp0 = jnp.int32(b * P + 1)
        base_pid = ss * SRNG
        lim = jnp.where(ss == NS - 1, jnp.int32(P - (NS - 1) * SRNG),
                        jnp.int32(SRNG))
        cbase = (cc * (NS * RPT)).astype(jnp.int32)

        pltpu.sync_copy(gxh.at[pl.ds(b * P + base_pid, SRNG + 32)], gxv)
        pltpu.sync_copy(gyh.at[pl.ds(b * P + base_pid, SRNG + 32)], gyv)

        plsc.store_scatter(bktf, [ids * 128], zi)

        def _route(ci, cnts):
            gyc = gyv[pl.ds(ci * L, L)]
            gxc = gxv[pl.ds(ci * L, L)]
            pidl = ci * L + ids
            m = (gyc >= cbase) & (gyc < cbase + NS * RPT) & (pidl < lim)
            posl = (gyc & (RPT - 1)) * W + gxc
            pk = posl * 32768 + (base_pid + pidl)
            ov = lax.shift_right_arithmetic(gyc, 4) - cc * NS
            new = []
            for o in range(NS):
                mo = m & (ov == o)
                plsc.store_compressed(
                    bktf.at[pl.ds(o * 128 + 1 + cnts[o], L)], pk, mask=mo)
                po = plsc.all_reduce_population_count(mo)
                new.append(jnp.minimum(cnts[o] + po[0], jnp.int32(111)))
            return tuple(new)

        with jax.named_scope("scan"):
            cnts = lax.fori_loop(0, SCH, _route,
                                 tuple(jnp.int32(0) for _ in range(NS)))

        cvec = zi
        for o in range(NS):
            cvec = jnp.where(ids == o, cnts[o], cvec)
        plsc.store_scatter(bktf, [ids * 128], cvec)
        plsc.subcore_barrier()
        pltpu.sync_copy(bktf, shm.at[pl.ds(ss * (NS * 128), NS * 128)])
        plsc.subcore_barrier()

        def _coll(t):
            return pltpu.make_async_copy(
                shm.at[pl.ds((t * NS + ss) * 128, 128)],
                tmpb.at[pl.ds(t * 128, 128)], semc)

        for t in range(NS):
            _coll(t).start()
        for t in range(NS):
            _coll(t).wait()

        counts = plsc.load_gather(tmpb, [ids * 128])
        incl = plsc.cumsum(counts)
        total = incl[NS - 1]

        with jax.named_scope("concat"):
            off = jnp.int32(0)
            for t in range(NS):
                for j in range(7):
                    cpk[pl.ds(off + j * L, L)] = tmpb[
                        pl.ds(t * 128 + 1 + j * L, L)]
                off = incl[t]

        def _dedup(k, carry):
            base = k * L
            ok = (base + ids) < total
            pk = cpk[pl.ds(base, L)]
            posl = lax.shift_right_arithmetic(pk, 15)
            key = posl * L + ids
            sk, _, sm = plsc.sort_key_val(key, key, mask=ok)
            pos_s = lax.shift_right_arithmetic(sk, 4)
            lane_s = sk & (L - 1)
            spos[:] = pos_s
            pkb[:] = pk
            nxt = plsc.load_gather(spos, [jnp.minimum(ids + 1, L - 1)])
            pk_s = plsc.load_gather(pkb, [lane_s])
            m_store = ((pos_s != nxt) | (ids == L - 1)) & sm
            val = stamp0 + (pk_s & 32767)
            plsc.store_scatter(wm, [pos_s], val, mask=m_store)
            return carry

        with jax.named_scope("dedup"):
            lax.fori_loop(0, lax.shift_right_arithmetic(total + L - 1, 4),
                          _dedup, 0)
        return total

    def do_ext(b, n):
        stamp0 = jnp.int32(b * P + 1)

        def _ext(j, nbs):
            nb0, nb1 = nbs
            base = j * L
            pk = cpk[pl.ds(base, L)]
            posc = lax.shift_right_arithmetic(pk, 15)
            idxc = pk & 32767
            ok = (base + ids) < n
            wv = plsc.load_gather(wm, [posc], mask=ok)
            win = ok & (wv == stamp0 + idxc)
            w0 = win & (posc < BCELLS)
            w1 = win & (posc >= BCELLS)
            plsc.store_compressed(bpk[0].at[pl.ds(nb0, L)], pk, mask=w0)
            plsc.store_compressed(bpk[1].at[pl.ds(nb1, L)],
                                  pk - BCELLS * 32768, mask=w1)
            c0 = plsc.all_reduce_population_count(w0)
            c1 = plsc.all_reduce_population_count(w1)
            return (jnp.minimum(nb0 + c0[0], jnp.int32(BCAP)),
                    jnp.minimum(nb1 + c1[0], jnp.int32(BCAP)))

        with jax.named_scope("ext"):
            return lax.fori_loop(
                0, lax.shift_right_arithmetic(n + L - 1, 4), _ext,
                (jnp.int32(0), jnp.int32(0)))

    def do_prep(b, band, nb):
        def _rc(k, carry):
            base = k * L
            pk = bpk[band][pl.ds(base, L)]
            posc = lax.shift_right_arithmetic(pk, 15)
            pid = pk & 32767
            ok = (base + ids) < nb
            rowl[band][pl.ds(base, L)] = lax.shift_right_arithmetic(
                posc, 9)
            coll[band][pl.ds(base, L)] = posc & (W - 1)
            hafl[band][pl.ds(base, L)] = (pid & 1) * C
            grow[band][pl.ds(base, L)] = jnp.where(
                ok, lax.shift_right_arithmetic(jnp.int32(b * P) + pid, 1),
                0)
            return carry

        with jax.named_scope("prep"):
            nch = lax.shift_left(
                lax.shift_right_arithmetic(nb + GCH - 1, GSH), GSH - 4)
            lax.fori_loop(0, nch, _rc, 0)

    def _g_dma(band, g):
        return pltpu.make_async_copy(
            feats.at[grow[band].at[pl.ds(g * GCH, GCH)]],
            winsrc.at[pl.ds(g * GCH, GCH), :], semg)

    def fire_g(band, nb):
        def _gf(g, carry):
            _g_dma(band, g).start()
            return carry

        lax.fori_loop(0, lax.shift_right_arithmetic(nb + GCH - 1, GSH),
                      _gf, 0)

    def wait_g(band, nb):
        def _gw(g, carry):
            _g_dma(band, g).wait()
            return carry

        with jax.named_scope("rowwait"):
            lax.fori_loop(0, lax.shift_right_arithmetic(nb + GCH - 1, GSH),
                          _gw, 0)

    def do_bands(b, band, nb):
        row0 = r0 + band * BAND
        nk = lax.shift_right_arithmetic(nb + L - 1, 4)

        def _insert2(u0, c0):
            def _ins(k, carry):
                base = k * L
                kv = base + ids
                ok = kv < nb
                rw = rowl[band][pl.ds(base, L)]
                cl = coll[band][pl.ds(base, L)]
                hf = hafl[band][pl.ds(base, L)]
                v0 = plsc.load_gather(winsrc, [kv, hf + c0], mask=ok)
                plsc.store_scatter(bands.at[u0], [rw, cl], v0, mask=ok)
                v1 = plsc.load_gather(winsrc, [kv, hf + (c0 + 1)], mask=ok)
                plsc.store_scatter(bands.at[u0 + 1], [rw, cl], v1, mask=ok)
                return carry

            lax.fori_loop(0, nk, _ins, 0)

        def _band_dma(u, c_tr, sem):
            return pltpu.make_async_copy(
                bands.at[u], out.at[b, c_tr, pl.ds(row0, BAND), :], sem)

        for pp in range(2):
            c0 = jnp.int32(2 * pp)
            _insert2(2 * pp, c0)
            _band_dma(2 * pp, c0, sems[2 * pp]).start()
            _band_dma(2 * pp + 1, c0 + 1, sems[2 * pp + 1]).start()

        def _grp(g2, carry):
            for pp in range(2):
                c0 = (g2 * 2 + pp) * 2
                u0 = 2 * pp
                _band_dma(u0, c0 - 4, sems[u0]).wait()
                _band_dma(u0 + 1, c0 - 3, sems[u0 + 1]).wait()
                _insert2(u0, c0)
                _band_dma(u0, c0, sems[u0]).start()
                _band_dma(u0 + 1, c0 + 1, sems[u0 + 1]).start()
            return carry

        with jax.named_scope("bands2"):
            lax.fori_loop(1, C // 4, _grp, 0)

        for u in range(4):
            _band_dma(u, jnp.int32(C - 4 + u), sems[u]).wait()

            def _rst(k, carry):
                base = k * L
                ok = (base + ids) < nb
                rw = rowl[band][pl.ds(base, L)]
                cl = coll[band][pl.ds(base, L)]
                plsc.store_scatter(bands.at[u], [rw, cl], zf, mask=ok)
                return carry

            with jax.named_scope("rst"):
                lax.fori_loop(0, nk, _rst, 0)

    n = do_scan(0)
    nb0, nb1 = do_ext(0, n)
    do_prep(0, 0, nb0)
    do_prep(0, 1, nb1)
    fire_g(0, nb0)
    for b in range(B):
        if b + 1 < B:
            n = do_scan(b + 1)
        wait_g(0, nb0)
        do_bands(b, 0, nb0)
        fire_g(1, nb1)
        wait_g(1, nb1)
        do_bands(b, 1, nb1)
        if b + 1 < B:
            nb0n, nb1n = do_ext(b + 1, n)
            do_prep(b + 1, 0, nb0n)
            do_prep(b + 1, 1, nb1n)
            fire_g(0, nb0n)
            nb0, nb1 = nb0n, nb1n


def kernel(pillar_features, pillar_coords):
    gx = pillar_coords[..., 0].reshape(-1).astype(jnp.int32)
    gy = pillar_coords[..., 1].reshape(-1).astype(jnp.int32)
    feats = pillar_features.reshape(B * P // 2, 2 * C)
    zb = jnp.zeros((BAND, W), jnp.float32)
    return _bev_sc(feats, gx, gy, zb)


# speedup vs baseline: 2.1847x; 1.2052x over previous
import functools

import jax
import jax.numpy as jnp
from jax import lax
from jax.experimental import pallas as pl
from jax.experimental.pallas import tpu as pltpu
from jax.experimental.pallas import tpu_sc as plsc

H = 512
W = 512
B = 4
P = 20000
C = 64
L = 16

_info = plsc.get_sparse_core_info()
NC = _info.num_cores
NS = _info.num_subcores
NT = NC * NS
RPT = H // NT
CELLS = RPT * W
BAND = 8
BCELLS = BAND * W
CAP = 2048
BCAP = 512
GCH = 8
GSH = 3
SRNG = 1248
SCH = 80

_mesh = plsc.VectorSubcoreMesh(core_axis_name="c", subcore_axis_name="s")


@functools.partial(
    pl.kernel,
    mesh=_mesh,
    compiler_params=pltpu.CompilerParams(needs_layout_passes=False),
    out_type=jax.ShapeDtypeStruct((B, C, H, W), jnp.float32),
    scratch_types=[
        pltpu.VMEM((SRNG + 32,), jnp.int32),
        pltpu.VMEM((SRNG + 32,), jnp.int32),
        pltpu.VMEM((NS * 128,), jnp.int32),
        pltpu.VMEM((NS * 128,), jnp.int32),
        pltpu.VMEM_SHARED((NS * NS * 128,), jnp.int32),
        pltpu.VMEM((CAP + L,), jnp.int32),
        pltpu.VMEM((BCAP + L,), jnp.int32),
        pltpu.VMEM((BCAP + L,), jnp.int32),
        pltpu.VMEM((CELLS,), jnp.int32),
        pltpu.VMEM((L,), jnp.int32),
        pltpu.VMEM((L,), jnp.int32),
        pltpu.VMEM((BCAP + L,), jnp.int32),
        pltpu.VMEM((BCAP + L,), jnp.int32),
        pltpu.VMEM((BCAP + L,), jnp.int32),
        pltpu.VMEM((BCAP + L,), jnp.int32),
        pltpu.VMEM((BCAP + L,), jnp.int32),
        pltpu.VMEM((BCAP + L,), jnp.int32),
        pltpu.VMEM((BCAP,), jnp.int32),
        pltpu.VMEM((BCAP,), jnp.int32),
        pltpu.VMEM((BCAP, 2 * C), jnp.float32),
        pltpu.VMEM((4, BAND, W), jnp.float32),
        pltpu.SemaphoreType.DMA,
        pltpu.SemaphoreType.DMA,
        pltpu.SemaphoreType.DMA,
        pltpu.SemaphoreType.DMA,
        pltpu.SemaphoreType.DMA,
        pltpu.SemaphoreType.DMA,
    ],
)
def _bev_sc(feats, gxh, gyh, zb, out, gxv, gyv, bktf, tmpb, shm, cpk,
            bpk0, bpk1, wm, spos, pkb, rowl0, rowl1, coll0, coll1, hafl0,
            hafl1, grow0, grow1, winsrc, bands,
            semc, semg, semb0, semb1, semb2, semb3):
    cc = lax.axis_index("c")
    ss = lax.axis_index("s")
    wid = cc * NS + ss
    r0 = (wid * RPT).astype(jnp.int32)
    ids = lax.iota(jnp.int32, L)
    zi = jnp.zeros((L,), jnp.int32)
    zf = jnp.zeros((L,), jnp.float32)
    sems = (semb0, semb1, semb2, semb3)
    bpk = (bpk0, bpk1)
    rowl = (rowl0, rowl1)
    coll = (coll0, coll1)
    hafl = (hafl0, hafl1)
    grow = (grow0, grow1)

    def _wm_body(j, carry):
        wm[pl.ds(j * L, L)] = zi
        return carry

    lax.fori_loop(0, CELLS // L, _wm_body, 0)
    for u in range(4):
        pltpu.sync_copy(zb, bands.at[u])

    def do_scan(b):
        stamp0 = jnp.int32(b * P + 1)
        base_pid = ss * SRNG
        lim = jnp.where(ss == NS - 1, jnp.int32(P - (NS - 1) * SRNG),
                        jnp.int32(SRNG))
        cbase = (cc * (NS * RPT)).astype(jnp.int32)

        pltpu.sync_copy(gxh.at[pl.ds(b * P + base_pid, SRNG + 32)], gxv)
        pltpu.sync_copy(gyh.at[pl.ds(b * P + base_pid, SRNG + 32)], gyv)

        plsc.store_scatter(bktf, [ids * 128], zi)

        def _route(ci, cnts):
            gyc = gyv[pl.ds(ci * L, L)]
            gxc = gxv[pl.ds(ci * L, L)]
            pidl = ci * L + ids
            m = (gyc >= cbase) & (gyc < cbase + NS * RPT) & (pidl < lim)
            posl = (gyc & (RPT - 1)) * W + gxc
            pk = posl * 32768 + (base_pid + pidl)
            ov = lax.shift_right_arithmetic(gyc, 4) - cc * NS
            new = []
            for o in range(NS):
                mo = m & (ov == o)
                plsc.store_compressed(
                    bktf.at[pl.ds(o * 128 + 1 + cnts[o], L)], pk, mask=mo)
                po = plsc.all_reduce_population_count(mo)
                new.append(jnp.minimum(cnts[o] + po[0], jnp.int32(111)))
            return tuple(new)

        with jax.named_scope("scan"):
            cnts = lax.fori_loop(0, SCH, _route,
                                 tuple(jnp.int32(0) for _ in range(NS)))

        cvec = zi
        for o in range(NS):
            cvec = jnp.where(ids == o, cnts[o], cvec)
        plsc.store_scatter(bktf, [ids * 128], cvec)
        plsc.subcore_barrier()
        pltpu.sync_copy(bktf, shm.at[pl.ds(ss * (NS * 128), NS * 128)])
        plsc.subcore_barrier()

        def _coll(t):
            return pltpu.make_async_copy(
                shm.at[pl.ds((t * NS + ss) * 128, 128)],
                tmpb.at[pl.ds(t * 128, 128)], semc)

        for t in range(NS):
            _coll(t).start()
        for t in range(NS):
            _coll(t).wait()

        counts = plsc.load_gather(tmpb, [ids * 128])
        incl = plsc.cumsum(counts)
        total = incl[NS - 1]

        with jax.named_scope("concat"):
            off = jnp.int32(0)
            for t in range(NS):
                for j in range(7):
                    cpk[pl.ds(off + j * L, L)] = tmpb[
                        pl.ds(t * 128 + 1 + j * L, L)]
                off = incl[t]

        def _dedup(k, carry):
            base = k * L
            ok = (base + ids) < total
            pk = cpk[pl.ds(base, L)]
            posl = lax.shift_right_arithmetic(pk, 15)
            key = posl * L + ids
            sk, _, sm = plsc.sort_key_val(key, key, mask=ok)
            pos_s = lax.shift_right_arithmetic(sk, 4)
            lane_s = sk & (L - 1)
            spos[:] = pos_s
            pkb[:] = pk
            nxt = plsc.load_gather(spos, [jnp.minimum(ids + 1, L - 1)])
            pk_s = plsc.load_gather(pkb, [lane_s])
            m_store = ((pos_s != nxt) | (ids == L - 1)) & sm
            val = stamp0 + (pk_s & 32767)
            plsc.store_scatter(wm, [pos_s], val, mask=m_store)
            return carry

        with jax.named_scope("dedup"):
            lax.fori_loop(0, lax.shift_right_arithmetic(total + L - 1, 4),
                          _dedup, 0)
        return total

    def do_ext(b, n):
        stamp0 = jnp.int32(b * P + 1)

        def _ext(j, nbs):
            nb0, nb1 = nbs
            base = j * L
            pk = cpk[pl.ds(base, L)]
            posc = lax.shift_right_arithmetic(pk, 15)
            idxc = pk & 32767
            ok = (base + ids) < n
            wv = plsc.load_gather(wm, [posc], mask=ok)
            win = ok & (wv == stamp0 + idxc)
            w0 = win & (posc < BCELLS)
            w1 = win & (posc >= BCELLS)
            plsc.store_compressed(bpk[0].at[pl.ds(nb0, L)], pk, mask=w0)
            plsc.store_compressed(bpk[1].at[pl.ds(nb1, L)],
                                  pk - BCELLS * 32768, mask=w1)
            c0 = plsc.all_reduce_population_count(w0)
            c1 = plsc.all_reduce_population_count(w1)
            return (jnp.minimum(nb0 + c0[0], jnp.int32(BCAP)),
                    jnp.minimum(nb1 + c1[0], jnp.int32(BCAP)))

        with jax.named_scope("ext"):
            return lax.fori_loop(
                0, lax.shift_right_arithmetic(n + L - 1, 4), _ext,
                (jnp.int32(0), jnp.int32(0)))

    def do_prep(b, band, nb):
        def _rc(k, carry):
            base = k * L
            pk = bpk[band][pl.ds(base, L)]
            posc = lax.shift_right_arithmetic(pk, 15)
            pid = pk & 32767
            ok = (base + ids) < nb
            rowl[band][pl.ds(base, L)] = lax.shift_right_arithmetic(
                posc, 9)
            coll[band][pl.ds(base, L)] = posc & (W - 1)
            hafl[band][pl.ds(base, L)] = (pid & 1) * C
            grow[band][pl.ds(base, L)] = jnp.where(
                ok, lax.shift_right_arithmetic(jnp.int32(b * P) + pid, 1),
                0)
            return carry

        with jax.named_scope("prep"):
            nwin = lax.shift_right_arithmetic(nb + GCH - 1, GSH)
            nch = lax.shift_right_arithmetic(nwin * GCH + L - 1, 4)
            lax.fori_loop(0, nch, _rc, 0)

    def _g_dma(band, g):
        return pltpu.make_async_copy(
            feats.at[grow[band].at[pl.ds(g * GCH, GCH)]],
            winsrc.at[pl.ds(g * GCH, GCH), :], semg)

    def fire_g(band, nb):
        def _gf(g, carry):
            _g_dma(band, g).start()
            return carry

        lax.fori_loop(0, lax.shift_right_arithmetic(nb + GCH - 1, GSH),
                      _gf, 0)

    def wait_g(band, nb):
        def _gw(g, carry):
            _g_dma(band, g).wait()
            return carry

        with jax.named_scope("rowwait"):
            lax.fori_loop(0, lax.shift_right_arithmetic(nb + GCH - 1, GSH),
                          _gw, 0)

    def do_bands(b, band, nb):
        row0 = r0 + band * BAND
        nk = lax.shift_right_arithmetic(nb + L - 1, 4)

        def _insert2(u0, c0):
            def _ins(k, carry):
                base = k * L
                kv = base + ids
                ok = kv < nb
                rw = rowl[band][pl.ds(base, L)]
                cl = coll[band][pl.ds(base, L)]
                hf = hafl[band][pl.ds(base, L)]
                v0 = plsc.load_gather(winsrc, [kv, hf + c0], mask=ok)
                plsc.store_scatter(bands.at[u0], [rw, cl], v0, mask=ok)
                v1 = plsc.load_gather(winsrc, [kv, hf + (c0 + 1)], mask=ok)
                plsc.store_scatter(bands.at[u0 + 1], [rw, cl], v1, mask=ok)
                return carry

            lax.fori_loop(0, nk, _ins, 0)

        def _band_dma(u, c_tr, sem):
            return pltpu.make_async_copy(
                bands.at[u], out.at[b, c_tr, pl.ds(row0, BAND), :], sem)

        for pp in range(2):
            c0 = jnp.int32(2 * pp)
            _insert2(2 * pp, c0)
            _band_dma(2 * pp, c0, sems[2 * pp]).start()
            _band_dma(2 * pp + 1, c0 + 1, sems[2 * pp + 1]).start()

        def _grp(g2, carry):
            for pp in range(2):
                c0 = (g2 * 2 + pp) * 2
                u0 = 2 * pp
                _band_dma(u0, c0 - 4, sems[u0]).wait()
                _band_dma(u0 + 1, c0 - 3, sems[u0 + 1]).wait()
                _insert2(u0, c0)
                _band_dma(u0, c0, sems[u0]).start()
                _band_dma(u0 + 1, c0 + 1, sems[u0 + 1]).start()
            return carry

        with jax.named_scope("bands2"):
            lax.fori_loop(1, C // 4, _grp, 0)

        for u in range(4):
            _band_dma(u, jnp.int32(C - 4 + u), sems[u]).wait()

            def _rst(k, carry):
                base = k * L
                ok = (base + ids) < nb
                rw = rowl[band][pl.ds(base, L)]
                cl = coll[band][pl.ds(base, L)]
                plsc.store_scatter(bands.at[u], [rw, cl], zf, mask=ok)
                return carry

            with jax.named_scope("rst"):
                lax.fori_loop(0, nk, _rst, 0)

    n = do_scan(0)
    nb0, nb1 = do_ext(0, n)
    do_prep(0, 0, nb0)
    do_prep(0, 1, nb1)
    fire_g(0, nb0)
    for b in range(B):
        if b + 1 < B:
            n = do_scan(b + 1)
        wait_g(0, nb0)
        do_bands(b, 0, nb0)
        fire_g(1, nb1)
        wait_g(1, nb1)
        do_bands(b, 1, nb1)
        if b + 1 < B:
            nb0n, nb1n = do_ext(b + 1, n)
            do_prep(b + 1, 0, nb0n)
            do_prep(b + 1, 1, nb1n)
            fire_g(0, nb0n)
            nb0, nb1 = nb0n, nb1n


def kernel(pillar_features, pillar_coords):
    gx = pillar_coords[..., 0].reshape(-1).astype(jnp.int32)
    gy = pillar_coords[..., 1].reshape(-1).astype(jnp.int32)
    feats = pillar_features.reshape(B * P // 2, 2 * C)
    zb = jnp.zeros((BAND, W), jnp.float32)
    return _bev_sc(feats, gx, gy, zb)
